# baseline (speedup 1.0000x reference)
import functools

import jax
import jax.numpy as jnp
from jax import lax
from jax.experimental import pallas as pl
from jax.experimental.pallas import tpu as pltpu
from jax.experimental.pallas import tpu_sc as plsc

NUM_TYPES = 10
D = 128
B_TOTAL = 16384 * 200
NC, NS, L = 2, 16, 16
NW = NC * NS
B_PER_W = B_TOTAL // NW
K = 2
C = K * 128
N_ITERS = B_PER_W // C


def _embed_kernel(x_hbm, table_hbm, out_hbm, pat_v, x_v, idx_v, rows_v, sem_x, sem_g):
    wid = lax.axis_index("s") * NC + lax.axis_index("c")
    base = wid * B_PER_W
    lane = jnp.arange(16, dtype=jnp.int32)

    def body(i, carry):
        row0 = base + i * C
        off = row0 * 8
        for r in range(C // 16):
            pat_v[pl.ds(r * 16, 16)] = (r * 16 + lane) * 8 + 7 + off
        cps = []
        for j in range(K):
            cps.append(
                pltpu.make_async_copy(
                    x_hbm.at[pat_v.at[pl.ds(j * 128, 128)]],
                    x_v.at[pl.ds(j * 128, 128)],
                    sem_x,
                )
            )
            cps[-1].start()
        for cp in cps:
            cp.wait()
        for r in range(C // 16):
            idx_v[pl.ds(r * 16, 16)] = x_v[pl.ds(r * 16, 16)].astype(jnp.int32)
        cps = []
        for j in range(K):
            cps.append(
                pltpu.make_async_copy(
                    table_hbm.at[idx_v.at[pl.ds(j * 128, 128)]],
                    rows_v.at[pl.ds(j * 128, 128)],
                    sem_g,
                )
            )
            cps[-1].start()
        for cp in cps:
            cp.wait()
        pltpu.sync_copy(rows_v, out_hbm.at[pl.ds(row0, C)])
        return carry

    lax.fori_loop(0, N_ITERS, body, 0)


@jax.jit
def kernel(x, table):
    x2 = x.reshape(B_TOTAL * 8)
    mesh = plsc.VectorSubcoreMesh(core_axis_name="c", subcore_axis_name="s")
    out = pl.kernel(
        _embed_kernel,
        mesh=mesh,
        out_type=jax.ShapeDtypeStruct((B_TOTAL, D), jnp.float32),
        scratch_types=[
            pltpu.VMEM((C,), jnp.int32),
            pltpu.VMEM((C,), jnp.float32),
            pltpu.VMEM((C,), jnp.int32),
            pltpu.VMEM((C, D), jnp.float32),
            pltpu.SemaphoreType.DMA,
            pltpu.SemaphoreType.DMA,
        ],
    )(x2, table)
    return out.reshape(16384, 200, D)
